# baseline (speedup 1.0000x reference)
import jax
import jax.numpy as jnp
from jax import lax
from jax.experimental import pallas as pl
from jax.experimental.pallas import tpu as pltpu
from jax.experimental.pallas import tpu_sc as plsc

_NC = 2
_NS = 16
_NW = _NC * _NS

_BATCH = 4096
_SEQ = 200
_D = 64
_B = _BATCH * _SEQ
_BPW = _B // _NW
_RPW = _BATCH // _NW
_RPC = 4
_CH = _RPC * _SEQ
_NBUF = 2
_NCHUNK = _RPW // _RPC
_NGROUP = _NCHUNK // _NBUF


def _sc_gather(idx_hbm, table_hbm, out_hbm, idx_all, rows_v, *sems):
    gsem = sems[:_NBUF]
    wsem = sems[_NBUF:]
    wid = lax.axis_index("s") * _NC + lax.axis_index("c")
    row_base = wid * _RPW

    pltpu.sync_copy(idx_hbm.at[pl.ds(wid * _BPW, _BPW)], idx_all)

    def group(q, carry):
        descs = []
        for b in range(_NBUF):
            g = q * _NBUF + b
            row0 = row_base + g * _RPC

            @pl.when(q > 0)
            def _():
                prev0 = row0 - _NBUF * _RPC
                for j in range(_RPC):
                    pltpu.make_async_copy(
                        rows_v.at[b, pl.ds(j * _SEQ, _SEQ)],
                        out_hbm.at[prev0 + j],
                        wsem[b],
                    ).wait()

            descs.append(
                pltpu.async_copy(
                    table_hbm.at[idx_all.at[pl.ds(g * _CH, _CH)]],
                    rows_v.at[b],
                    gsem[b],
                )
            )
        for b in range(_NBUF):
            g = q * _NBUF + b
            row0 = row_base + g * _RPC
            descs[b].wait()
            for j in range(_RPC):
                pltpu.async_copy(
                    rows_v.at[b, pl.ds(j * _SEQ, _SEQ)],
                    out_hbm.at[row0 + j],
                    wsem[b],
                )
        return carry

    lax.fori_loop(0, _NGROUP, group, 0)

    for b in range(_NBUF):
        g = (_NGROUP - 1) * _NBUF + b
        row0 = row_base + g * _RPC
        for j in range(_RPC):
            pltpu.make_async_copy(
                rows_v.at[b, pl.ds(j * _SEQ, _SEQ)],
                out_hbm.at[row0 + j],
                wsem[b],
            ).wait()


@jax.jit
def kernel(token_ids, table):
    idx = token_ids.reshape(-1).astype(jnp.int32)
    mesh = plsc.VectorSubcoreMesh(
        core_axis_name="c", subcore_axis_name="s",
        num_cores=_NC, num_subcores=_NS,
    )
    out = pl.kernel(
        _sc_gather,
        out_type=jax.ShapeDtypeStruct((_BATCH, _SEQ, _D), jnp.float32),
        mesh=mesh,
        scratch_types=(
            [pltpu.VMEM((_BPW,), jnp.int32),
             pltpu.VMEM((_NBUF, _CH, _D), jnp.float32)]
            + [pltpu.SemaphoreType.DMA] * (2 * _NBUF)
        ),
        compiler_params=pltpu.CompilerParams(use_tc_tiling_on_sc=False),
    )(idx, table)
    return out


# speedup vs baseline: 1.0068x; 1.0068x over previous
import jax
import jax.numpy as jnp
from jax import lax
from jax.experimental import pallas as pl
from jax.experimental.pallas import tpu as pltpu
from jax.experimental.pallas import tpu_sc as plsc

_NC = 2
_NS = 16
_NW = _NC * _NS

_BATCH = 4096
_SEQ = 200
_D = 64
_B = _BATCH * _SEQ
_BPW = _B // _NW
_RPW = _BATCH // _NW
_RPC = 2
_CH = _RPC * _SEQ
_NBUF = 4
_NCHUNK = _RPW // _RPC
_NGROUP = _NCHUNK // _NBUF


def _sc_gather(idx_hbm, table_hbm, out_hbm, idx_all, rows_v, *sems):
    gsem = sems[:_NBUF]
    wsem = sems[_NBUF:]
    wid = lax.axis_index("s") * _NC + lax.axis_index("c")
    row_base = wid * _RPW

    pltpu.sync_copy(idx_hbm.at[pl.ds(wid * _BPW, _BPW)], idx_all)

    def group(q, carry):
        descs = []
        for b in range(_NBUF):
            g = q * _NBUF + b
            row0 = row_base + g * _RPC

            @pl.when(q > 0)
            def _():
                prev0 = row0 - _NBUF * _RPC
                for j in range(_RPC):
                    pltpu.make_async_copy(
                        rows_v.at[b, pl.ds(j * _SEQ, _SEQ)],
                        out_hbm.at[prev0 + j],
                        wsem[b],
                    ).wait()

            descs.append(
                pltpu.async_copy(
                    table_hbm.at[idx_all.at[pl.ds(g * _CH, _CH)]],
                    rows_v.at[b],
                    gsem[b],
                )
            )
        for b in range(_NBUF):
            g = q * _NBUF + b
            row0 = row_base + g * _RPC
            descs[b].wait()
            for j in range(_RPC):
                pltpu.async_copy(
                    rows_v.at[b, pl.ds(j * _SEQ, _SEQ)],
                    out_hbm.at[row0 + j],
                    wsem[b],
                )
        return carry

    lax.fori_loop(0, _NGROUP, group, 0)

    for b in range(_NBUF):
        g = (_NGROUP - 1) * _NBUF + b
        row0 = row_base + g * _RPC
        for j in range(_RPC):
            pltpu.make_async_copy(
                rows_v.at[b, pl.ds(j * _SEQ, _SEQ)],
                out_hbm.at[row0 + j],
                wsem[b],
            ).wait()


@jax.jit
def kernel(token_ids, table):
    idx = token_ids.reshape(-1).astype(jnp.int32)
    mesh = plsc.VectorSubcoreMesh(
        core_axis_name="c", subcore_axis_name="s",
        num_cores=_NC, num_subcores=_NS,
    )
    out = pl.kernel(
        _sc_gather,
        out_type=jax.ShapeDtypeStruct((_BATCH, _SEQ, _D), jnp.float32),
        mesh=mesh,
        scratch_types=(
            [pltpu.VMEM((_BPW,), jnp.int32),
             pltpu.VMEM((_NBUF, _CH, _D), jnp.float32)]
            + [pltpu.SemaphoreType.DMA] * (2 * _NBUF)
        ),
        compiler_params=pltpu.CompilerParams(use_tc_tiling_on_sc=False),
    )(idx, table)
    return out
